# baseline (speedup 1.0000x reference)
import functools

import jax
import jax.numpy as jnp
from jax import lax
from jax.experimental import pallas as pl
from jax.experimental.pallas import tpu as pltpu
from jax.experimental.pallas import tpu_sc as plsc

BATCH = 16384
EMBED = 64
NC = 2
NS = 16
NW = NC * NS
RPW = BATCH // NW
GATHER_CHUNK = 128
NCHUNK = RPW // GATHER_CHUNK
GROUPS = RPW // 16
EPS2 = 1e-24


def _rsqrt_newton(x):
    i = plsc.bitcast(x, jnp.int32)
    i = jnp.int32(0x5F3759DF) - (i >> 1)
    y = plsc.bitcast(i, jnp.float32)
    for _ in range(3):
        y = y * (1.5 - 0.5 * x * y * y)
    return y


def _make_sc_kernel():
    mesh = plsc.VectorSubcoreMesh(core_axis_name="c", subcore_axis_name="s")

    @functools.partial(
        pl.kernel,
        mesh=mesh,
        out_type=jax.ShapeDtypeStruct((NC, 16), jnp.float32),
        scratch_types=[
            pltpu.VMEM((NCHUNK, GATHER_CHUNK), jnp.int32),
            pltpu.VMEM((RPW, EMBED), jnp.float32),
            pltpu.VMEM((RPW, EMBED), jnp.float32),
            pltpu.VMEM((16,), jnp.float32),
            pltpu.VMEM((NS, 16), jnp.float32),
            pltpu.VMEM_SHARED((NS, 16), jnp.float32),
            pltpu.SemaphoreType.DMA,
        ],
        compiler_params=pltpu.CompilerParams(
            needs_layout_passes=False, use_tc_tiling_on_sc=False
        ),
    )
    def sc_kernel(feat_hbm, lab_hbm, cent_hbm, out_hbm,
                  lab_v, feat_v, cent_v, part_v, red_v, shared, sem):
        cid = lax.axis_index("c")
        sid = lax.axis_index("s")
        wid = sid * NC + cid
        base = wid * RPW

        pltpu.sync_copy(lab_hbm.at[pl.ds(wid * NCHUNK, NCHUNK)], lab_v)
        copies = [
            pltpu.async_copy(
                cent_hbm.at[lab_v.at[j]],
                cent_v.at[pl.ds(j * GATHER_CHUNK, GATHER_CHUNK)],
                sem,
            )
            for j in range(NCHUNK)
        ]
        pltpu.sync_copy(feat_hbm.at[pl.ds(base, RPW)], feat_v)
        for cp in copies:
            cp.wait()

        lanes = lax.iota(jnp.int32, 16)

        def group_body(g, acc):
            rows = g * 16 + lanes
            d = None
            for e in range(EMBED):
                col = jnp.full((16,), e, jnp.int32)
                fv = plsc.load_gather(feat_v, [rows, col])
                cv = plsc.load_gather(cent_v, [rows, col])
                if d is None:
                    d = fv * cv
                    ff = fv * fv
                    cc = cv * cv
                else:
                    d = d + fv * cv
                    ff = ff + fv * fv
                    cc = cc + cv * cv
            rf = _rsqrt_newton(jnp.maximum(ff, EPS2))
            rc = _rsqrt_newton(jnp.maximum(cc, EPS2))
            cos = d * rf * rc
            cos = jnp.minimum(jnp.maximum(cos, -1.0), 1.0)
            return acc + (1.0 - cos)

        acc = lax.fori_loop(0, GROUPS, group_body, jnp.zeros((16,), jnp.float32))
        part_v[...] = acc * (1.0 / BATCH)

        pltpu.sync_copy(part_v, shared.at[sid])
        plsc.subcore_barrier()

        @pl.when(sid == 0)
        def _():
            pltpu.sync_copy(shared, red_v)
            tot = red_v[0]
            for i in range(1, NS):
                tot = tot + red_v[i]
            total = jnp.sum(tot, axis=0)
            part_v[...] = jnp.broadcast_to(total, (16,))
            pltpu.sync_copy(part_v, out_hbm.at[cid])

    return sc_kernel


_SC_KERNEL = _make_sc_kernel()


def kernel(features, labels, centers):
    lab2d = labels.astype(jnp.int32).reshape(NW * NCHUNK, GATHER_CHUNK)
    out = _SC_KERNEL(features, lab2d, centers)
    return out[0, 0] + out[1, 0]


# speedup vs baseline: 1.0120x; 1.0120x over previous
import functools

import jax
import jax.numpy as jnp
from jax import lax
from jax.experimental import pallas as pl
from jax.experimental.pallas import tpu as pltpu
from jax.experimental.pallas import tpu_sc as plsc

BATCH = 16384
EMBED = 64
NC = 2
NS = 16
NW = NC * NS
RPW = BATCH // NW
GROUPS = RPW // 16
EPS2 = 1e-24


def _rsqrt_newton(x):
    i = plsc.bitcast(x, jnp.int32)
    i = jnp.int32(0x5F3759DF) - (i >> 1)
    y = plsc.bitcast(i, jnp.float32)
    for _ in range(3):
        y = y * (1.5 - 0.5 * x * y * y)
    return y


def _make_sc_kernel():
    mesh = plsc.VectorSubcoreMesh(core_axis_name="c", subcore_axis_name="s")

    @functools.partial(
        pl.kernel,
        mesh=mesh,
        out_type=jax.ShapeDtypeStruct((NC * 8, 128), jnp.float32),
        scratch_types=[
            pltpu.VMEM((8, 128), jnp.int32),
            pltpu.VMEM((RPW // 2, 128), jnp.float32),
            pltpu.VMEM((RPW, EMBED), jnp.float32),
            pltpu.VMEM((8, 128), jnp.float32),
            pltpu.VMEM((NS * 8, 128), jnp.float32),
            pltpu.VMEM_SHARED((NS * 8, 128), jnp.float32),
            pltpu.SemaphoreType.DMA,
        ],
        compiler_params=pltpu.CompilerParams(
            needs_layout_passes=False, use_tc_tiling_on_sc=True
        ),
    )
    def sc_kernel(feat_hbm, lab_hbm, cent_hbm, out_hbm,
                  lab_v, feat_v, cent_v, part_v, red_v, shared, sem):
        cid = lax.axis_index("c")
        sid = lax.axis_index("s")
        wid = sid * NC + cid

        fbase = pl.multiple_of(wid * (RPW // 2), RPW // 2)
        pltpu.sync_copy(feat_hbm.at[pl.ds(fbase, RPW // 2)], feat_v)

        blk = pl.multiple_of((wid // 2) * 8, 8)
        rowoff = (wid % 2) * 4
        pltpu.sync_copy(lab_hbm.at[pl.ds(blk, 8)], lab_v)

        lanes = lax.iota(jnp.int32, 16)

        def gather_group(g, carry):
            lvec = lab_v[rowoff + g // 8, pl.ds((g % 8) * 16, 16)]
            for kk in range(16):
                l = jnp.sum(jnp.where(lanes == kk, lvec, 0))
                pltpu.async_copy(
                    cent_hbm.at[pl.ds(l, 1)],
                    cent_v.at[pl.ds(g * 16 + kk, 1)],
                    sem,
                ).wait()
            return carry

        lax.fori_loop(0, GROUPS, gather_group, 0)

        def group_body(g, acc):
            rows = g * 16 + lanes
            frow = rows >> 1
            fcolbase = (rows & 1) * EMBED
            d = None
            for e in range(EMBED):
                col = jnp.full((16,), e, jnp.int32)
                fv = plsc.load_gather(feat_v, [frow, fcolbase + e])
                cv = plsc.load_gather(cent_v, [rows, col])
                if d is None:
                    d = fv * cv
                    ff = fv * fv
                    cc = cv * cv
                else:
                    d = d + fv * cv
                    ff = ff + fv * fv
                    cc = cc + cv * cv
            rf = _rsqrt_newton(jnp.maximum(ff, EPS2))
            rc = _rsqrt_newton(jnp.maximum(cc, EPS2))
            cos = d * rf * rc
            cos = jnp.minimum(jnp.maximum(cos, -1.0), 1.0)
            return acc + (1.0 - cos)

        acc = lax.fori_loop(0, GROUPS, group_body, jnp.zeros((16,), jnp.float32))
        part_v[0, pl.ds(0, 16)] = acc * (1.0 / BATCH)

        sblk = pl.multiple_of(sid * 8, 8)
        pltpu.sync_copy(part_v, shared.at[pl.ds(sblk, 8)])
        plsc.subcore_barrier()

        @pl.when(sid == 0)
        def _():
            pltpu.sync_copy(shared, red_v)
            tot = red_v[0, pl.ds(0, 16)]
            for i in range(1, NS):
                tot = tot + red_v[i * 8, pl.ds(0, 16)]
            total = jnp.sum(tot, axis=0)
            part_v[0, pl.ds(0, 16)] = jnp.broadcast_to(total, (16,))
            cblk = pl.multiple_of(cid * 8, 8)
            pltpu.sync_copy(part_v, out_hbm.at[pl.ds(cblk, 8)])

    return sc_kernel


_SC_KERNEL = _make_sc_kernel()


def kernel(features, labels, centers):
    feat2d = features.reshape(BATCH // 2, 128)
    lab2d = labels.astype(jnp.int32).reshape(128, 128)
    out = _SC_KERNEL(feat2d, lab2d, centers)
    return out[0, 0] + out[8, 0]


# speedup vs baseline: 1.5139x; 1.4960x over previous
import functools

import jax
import jax.numpy as jnp
from jax import lax
from jax.experimental import pallas as pl
from jax.experimental.pallas import tpu as pltpu
from jax.experimental.pallas import tpu_sc as plsc

BATCH = 16384
EMBED = 64
NC = 2
NS = 16
NW = NC * NS
RPW = BATCH // NW
GROUPS = RPW // 16
EPS2 = 1e-24


def _rsqrt_newton(x):
    i = plsc.bitcast(x, jnp.int32)
    i = jnp.int32(0x5F3759DF) - (i >> 1)
    y = plsc.bitcast(i, jnp.float32)
    for _ in range(3):
        y = y * (1.5 - 0.5 * x * y * y)
    return y


def _make_sc_kernel():
    mesh = plsc.VectorSubcoreMesh(core_axis_name="c", subcore_axis_name="s")

    @functools.partial(
        pl.kernel,
        mesh=mesh,
        out_type=jax.ShapeDtypeStruct((NC * 8, 128), jnp.float32),
        scratch_types=[
            pltpu.VMEM((8, 128), jnp.int32),
            pltpu.VMEM((RPW // 2, 128), jnp.float32),
            pltpu.VMEM((RPW, EMBED), jnp.float32),
            pltpu.VMEM((8, 128), jnp.float32),
            pltpu.VMEM((NS * 8, 128), jnp.float32),
            pltpu.VMEM_SHARED((NS * 8, 128), jnp.float32),
            pltpu.SemaphoreType.DMA,
        ],
        compiler_params=pltpu.CompilerParams(
            needs_layout_passes=False, use_tc_tiling_on_sc=True
        ),
    )
    def sc_kernel(feat_hbm, lab_hbm, cent_hbm, out_hbm,
                  lab_v, feat_v, cent_v, part_v, red_v, shared, sem):
        cid = lax.axis_index("c")
        sid = lax.axis_index("s")
        wid = sid * NC + cid

        fbase = pl.multiple_of(wid * (RPW // 2), RPW // 2)
        pltpu.sync_copy(feat_hbm.at[pl.ds(fbase, RPW // 2)], feat_v)

        blk = pl.multiple_of((wid // 2) * 8, 8)
        rowoff = (wid % 2) * 4
        pltpu.sync_copy(lab_hbm.at[pl.ds(blk, 8)], lab_v)

        lanes = lax.iota(jnp.int32, 16)

        def gather_round(j, carry):
            pos = j + 32 * lanes
            lvec = plsc.load_gather(lab_v, [rowoff + pos // 128, pos % 128])
            copies = []
            for k in range(16):
                l = jnp.sum(jnp.where(lanes == k, lvec, 0))
                copies.append(pltpu.async_copy(
                    cent_hbm.at[pl.ds(l, 1)],
                    cent_v.at[pl.ds(k * 32 + j, 1)],
                    sem,
                ))
            for cp in copies:
                cp.wait()
            return carry

        lax.fori_loop(0, 32, gather_round, 0)

        def group_body(g, acc):
            rows = g * 16 + lanes
            frow = rows >> 1
            fcolbase = (rows & 1) * EMBED
            d = None
            for e in range(EMBED):
                col = jnp.full((16,), e, jnp.int32)
                fv = plsc.load_gather(feat_v, [frow, fcolbase + e])
                cv = plsc.load_gather(cent_v, [rows, col])
                if d is None:
                    d = fv * cv
                    ff = fv * fv
                    cc = cv * cv
                else:
                    d = d + fv * cv
                    ff = ff + fv * fv
                    cc = cc + cv * cv
            rf = _rsqrt_newton(jnp.maximum(ff, EPS2))
            rc = _rsqrt_newton(jnp.maximum(cc, EPS2))
            cos = d * rf * rc
            cos = jnp.minimum(jnp.maximum(cos, -1.0), 1.0)
            return acc + (1.0 - cos)

        acc = lax.fori_loop(0, GROUPS, group_body, jnp.zeros((16,), jnp.float32))
        part_v[0, pl.ds(0, 16)] = acc * (1.0 / BATCH)

        sblk = pl.multiple_of(sid * 8, 8)
        pltpu.sync_copy(part_v, shared.at[pl.ds(sblk, 8)])
        plsc.subcore_barrier()

        @pl.when(sid == 0)
        def _():
            pltpu.sync_copy(shared, red_v)
            tot = red_v[0, pl.ds(0, 16)]
            for i in range(1, NS):
                tot = tot + red_v[i * 8, pl.ds(0, 16)]
            total = jnp.sum(tot, axis=0)
            part_v[0, pl.ds(0, 16)] = jnp.broadcast_to(total, (16,))
            cblk = pl.multiple_of(cid * 8, 8)
            pltpu.sync_copy(part_v, out_hbm.at[pl.ds(cblk, 8)])

    return sc_kernel


_SC_KERNEL = _make_sc_kernel()


def kernel(features, labels, centers):
    feat2d = features.reshape(BATCH // 2, 128)
    lab2d = labels.astype(jnp.int32).reshape(128, 128)
    out = _SC_KERNEL(feat2d, lab2d, centers)
    return out[0, 0] + out[8, 0]
